# baseline (speedup 1.0000x reference)
import functools

import numpy as np
import jax
import jax.numpy as jnp
from jax import lax
from jax.experimental import pallas as pl
from jax.experimental.pallas import tpu as pltpu
from jax.experimental.pallas import tpu_sc as plsc

NUM_TABLES = 26
VOCAB = 100000
DIM = 64
HIST = 200
LANES = 16
NTILES = 32
WIN = 128

K_SC = 13
K_TC = NUM_TABLES - K_SC
NLOOK = K_SC * HIST
GRP = 8
PER_TILE = 88
NGRP = PER_TILE // GRP
IDX_PAD = PER_TILE * NTILES + LANES

RING = 16

_NI = NUM_TABLES + 1
_PAIRS = [(i, j) for i in range(_NI) for j in range(i)]
NPAIR = len(_PAIRS)

_SX = np.zeros((NPAIR, NUM_TABLES), np.float32)
_A2 = np.zeros((NPAIR, NUM_TABLES), np.float32)
_B2T = np.zeros((NPAIR, NUM_TABLES), np.float32)
for _p, (_i, _j) in enumerate(_PAIRS):
    if _j == 0:
        _SX[_p, _i - 1] = 1.0
    else:
        _A2[_p, _i - 1] = 1.0
        _B2T[_p, _j - 1] = 1.0

_CSC = np.zeros((NUM_TABLES, 2 * NTILES), np.float32)
for _t in range(NTILES):
    if PER_TILE * _t >= NLOOK:
        continue
    _k0 = min((PER_TILE * _t) // HIST, K_SC - 1)
    _CSC[_k0, 2 * _t] = 1.0
    _CSC[min(_k0 + 1, K_SC - 1), 2 * _t + 1] = 1.0

_CTC = np.zeros((NUM_TABLES, K_TC), np.float32)
for _j in range(K_TC):
    _CTC[K_SC + _j, _j] = 1.0


def _bags_body(idx_hbm, table_hbm, out_hbm, idx_v, blk, ly_v, sem):
    t = lax.axis_index("s") * 2 + lax.axis_index("c")
    q0 = t * PER_TILE

    @pl.when(q0 < NLOOK)
    def _():
        pltpu.sync_copy(
            idx_hbm.at[pl.ds(pl.multiple_of(q0, 8), PER_TILE + LANES)], idx_v)
        iota16 = lax.iota(jnp.int32, LANES)
        k0 = jnp.minimum(q0 // HIST, K_SC - 1)

        def group(g, acc):
            base = pl.multiple_of(g * GRP, GRP)
            vv = idx_v[pl.ds(base, LANES)]
            lanes = []
            for s in range(GRP):
                off = pl.multiple_of((vv[s] // WIN) * WIN, WIN)
                q = q0 + base + s
                k = jnp.minimum(q // HIST, K_SC - 1)
                lanes.append((vv[s] - off, k - k0, q < NLOOK))
                pltpu.make_async_copy(
                    table_hbm.at[k, :, pl.ds(off, WIN)], blk.at[s], sem).start()
            for s in range(GRP):
                pltpu.make_async_copy(
                    table_hbm.at[k0, :, pl.ds(0, WIN)], blk.at[s], sem).wait()
            for s in range(GRP):
                lane, slot, live = lanes[s]
                w0 = jnp.where(live & (slot == 0), 1.0, 0.0)
                w1 = jnp.where(live & (slot == 1), 1.0, 0.0)
                w0v = jnp.full((LANES,), w0, jnp.float32)
                w1v = jnp.full((LANES,), w1, jnp.float32)
                lanev = jnp.full((LANES,), lane, jnp.int32)
                slotv = jnp.full((LANES,), s, jnp.int32)
                new = []
                for j in range(DIM // LANES):
                    g_ = plsc.load_gather(blk, [slotv, iota16 + 16 * j, lanev])
                    new.append(acc[j] + g_ * w0v)
                    new.append(acc[DIM // LANES + j] + g_ * w1v)
                acc = tuple(new[::2]) + tuple(new[1::2])
            return acc

        acc = lax.fori_loop(
            0, NGRP, group,
            tuple(jnp.zeros((LANES,), jnp.float32)
                  for _ in range(2 * (DIM // LANES))))
        for u in range(2):
            for j in range(DIM // LANES):
                ly_v[u, pl.ds(j * LANES, LANES)] = acc[u * (DIM // LANES) + j]
        pltpu.sync_copy(ly_v, out_hbm.at[t])


def _bags(idx_flat, emb):
    mesh = plsc.VectorSubcoreMesh(core_axis_name="c", subcore_axis_name="s")
    f = functools.partial(
        pl.kernel,
        out_type=jax.ShapeDtypeStruct((NTILES, 2, DIM), jnp.float32),
        mesh=mesh,
        scratch_types=[
            pltpu.VMEM((PER_TILE + LANES,), jnp.int32),
            pltpu.VMEM((GRP, DIM, WIN), jnp.float32),
            pltpu.VMEM((2, DIM), jnp.float32),
            pltpu.SemaphoreType.DMA,
        ],
        compiler_params=pltpu.CompilerParams(needs_layout_passes=False),
    )(_bags_body)
    return f(idx_flat, emb)


def _tcg_body(idx_sm, emb_hbm, out_ref, buf, sem):
    lane_iota = lax.broadcasted_iota(jnp.int32, (1, WIN), 1)

    def fire(k, i, slot):
        v = idx_sm[k, i]
        off = pl.multiple_of((v // WIN) * WIN, WIN)
        pltpu.make_async_copy(
            emb_hbm.at[K_SC + k, :, pl.ds(off, WIN)], buf.at[slot],
            sem.at[slot]).start()

    for kk in range(K_TC):
        for r in range(RING):
            fire(kk, r, r)

        def body(i, acc):
            slot = lax.rem(i, RING)
            v = idx_sm[kk, i]
            off = (v // WIN) * WIN
            pltpu.make_async_copy(
                emb_hbm.at[K_SC + kk, :, pl.ds(0, WIN)], buf.at[slot],
                sem.at[slot]).wait()
            block = buf[slot]
            mask = (lane_iota == (v - off)).astype(jnp.float32)
            acc = acc + block * mask

            @pl.when(i + RING < HIST)
            def _():
                fire(kk, i + RING, slot)
            return acc

        acc = lax.fori_loop(0, HIST, body,
                            jnp.zeros((DIM, WIN), jnp.float32))
        out_ref[kk] = jnp.sum(acc, axis=1)


def _tc_gather(idx_tc, emb):
    return pl.pallas_call(
        _tcg_body,
        out_shape=jax.ShapeDtypeStruct((K_TC, DIM), jnp.float32),
        in_specs=[
            pl.BlockSpec(memory_space=pltpu.SMEM),
            pl.BlockSpec(memory_space=pltpu.MemorySpace.HBM),
        ],
        scratch_shapes=[
            pltpu.VMEM((RING, DIM, WIN), jnp.float32),
            pltpu.SemaphoreType.DMA((RING,)),
        ],
    )(idx_tc, emb)


def _mv(W, x):
    return lax.dot_general(W, x, (((1,), (0,)), ((), ())),
                           preferred_element_type=jnp.float32)


def _mlp_body(x_ref, parts_ref, lytc_ref, csc_ref, ctc_ref,
              sx_ref, a2_ref, b2t_ref,
              bw0, bb0, bw1, bb1, bw2, bb2,
              tw0x, tw0z, tb0, tw1, tb1, tw2, tb2, out_ref):
    x = x_ref[...]
    x = jnp.maximum(_mv(bw0[...], x) + bb0[...], 0.0)
    x = jnp.maximum(_mv(bw1[...], x) + bb1[...], 0.0)
    x = jnp.maximum(_mv(bw2[...], x) + bb2[...], 0.0)

    L = lax.dot_general(csc_ref[...], parts_ref[...], (((1,), (0,)), ((), ())),
                        preferred_element_type=jnp.float32)
    L = L + lax.dot_general(ctc_ref[...], lytc_ref[...], (((1,), (0,)), ((), ())),
                            preferred_element_type=jnp.float32)
    zx = _mv(L, x)
    ZL = lax.dot_general(L, L, (((1,), (1,)), ((), ())),
                         preferred_element_type=jnp.float32)
    zf = _mv(sx_ref[...], zx)
    ZA = lax.dot_general(a2_ref[...], ZL, (((1,), (0,)), ((), ())),
                         preferred_element_type=jnp.float32)
    zf = zf + jnp.sum(ZA * b2t_ref[...], axis=1, keepdims=True)

    h = jnp.maximum(_mv(tw0x[...], x) + _mv(tw0z[...], zf) + tb0[...], 0.0)
    h = jnp.maximum(_mv(tw1[...], h) + tb1[...], 0.0)
    h = _mv(tw2[...], h) + tb2[...]
    out_ref[...] = jax.nn.sigmoid(h)


def kernel(dense_x, lS_i, emb, bot_Ws, bot_bs, top_Ws, top_bs):
    idx = lS_i[:, 0, :]
    idx_flat = jnp.pad(idx[:K_SC].reshape(-1), (0, IDX_PAD - NLOOK))
    emb_t = emb.transpose(0, 2, 1)
    parts = _bags(idx_flat, emb_t)
    parts = parts.reshape(2 * NTILES, DIM)
    ly_tc = _tc_gather(idx[K_SC:], emb_t)

    x0 = dense_x.reshape(13, 1)
    csc = jnp.asarray(_CSC)
    ctc = jnp.asarray(_CTC)
    sx = jnp.asarray(_SX)
    a2 = jnp.asarray(_A2)
    b2t = jnp.asarray(_B2T)
    bb = [b.reshape(-1, 1) for b in bot_bs]
    tb = [b.reshape(-1, 1) for b in top_bs]
    tw0x = top_Ws[0][:, :DIM]
    tw0z = top_Ws[0][:, DIM:]

    return pl.pallas_call(
        _mlp_body,
        out_shape=jax.ShapeDtypeStruct((1, 1), jnp.float32),
    )(x0, parts, ly_tc, csc, ctc, sx, a2, b2t,
      bot_Ws[0], bb[0], bot_Ws[1], bb[1], bot_Ws[2], bb[2],
      tw0x, tw0z, tb[0], top_Ws[1], tb[1], top_Ws[2], tb[2])


# speedup vs baseline: 1.9695x; 1.9695x over previous
import functools

import numpy as np
import jax
import jax.numpy as jnp
from jax import lax
from jax.experimental import pallas as pl
from jax.experimental.pallas import tpu as pltpu
from jax.experimental.pallas import tpu_sc as plsc

NUM_TABLES = 26
VOCAB = 100000
DIM = 64
HIST = 200
LANES = 16
NTILES = 32
NLOOK = NUM_TABLES * HIST
PER_TILE = 168
GRP = 7
WIN = 128
NSB = 3
IDX_PAD = PER_TILE * NTILES + LANES
_AOFF = [((u * 7) // 8) * 8 for u in range(8)]
_OIN = [(u * 7) % 8 for u in range(8)]

_NI = NUM_TABLES + 1
_PAIRS = [(i, j) for i in range(_NI) for j in range(i)]
NPAIR = len(_PAIRS)

_SX = np.zeros((NPAIR, NUM_TABLES), np.float32)
_A2 = np.zeros((NPAIR, NUM_TABLES), np.float32)
_B2T = np.zeros((NPAIR, NUM_TABLES), np.float32)
for _p, (_i, _j) in enumerate(_PAIRS):
    if _j == 0:
        _SX[_p, _i - 1] = 1.0
    else:
        _A2[_p, _i - 1] = 1.0
        _B2T[_p, _j - 1] = 1.0

_COMB = np.zeros((NUM_TABLES, 2 * NTILES), np.float32)
for _t in range(NTILES):
    if PER_TILE * _t >= NLOOK:
        continue
    _k0 = min((PER_TILE * _t) // HIST, NUM_TABLES - 1)
    _COMB[_k0, 2 * _t] = 1.0
    _COMB[min(_k0 + 1, NUM_TABLES - 1), 2 * _t + 1] = 1.0


def _bags_body(idx_hbm, table_hbm, out_hbm, idx_v, blk, ly_v, sem):
    t = lax.axis_index("s") * 2 + lax.axis_index("c")
    q0 = t * PER_TILE

    @pl.when(q0 < NLOOK)
    def _():
        pltpu.sync_copy(
            idx_hbm.at[pl.ds(pl.multiple_of(q0, 8), PER_TILE + LANES)], idx_v)
        iota16 = lax.iota(jnp.int32, LANES)
        k0 = jnp.minimum(q0 // HIST, NUM_TABLES - 1)

        def loadvv(sb, u):
            lo = pl.multiple_of(sb * 56 + _AOFF[u], 8)
            return idx_v[pl.ds(lo, LANES)]

        def fire(sb, u):
            p = u % 2
            vv = loadvv(sb, u)
            for s in range(GRP):
                v = vv[_OIN[u] + s]
                off = pl.multiple_of((v // WIN) * WIN, WIN)
                q = q0 + sb * 56 + u * 7 + s
                k = jnp.minimum(q // HIST, NUM_TABLES - 1)
                pltpu.make_async_copy(
                    table_hbm.at[k, :, pl.ds(off, WIN)], blk.at[p, s],
                    sem.at[p]).start()

        def extract(sb, u, acc):
            p = u % 2
            vv = loadvv(sb, u)
            pv = jnp.full((LANES,), p, jnp.int32)
            for s in range(GRP):
                v = vv[_OIN[u] + s]
                lane = v - (v // WIN) * WIN
                q = q0 + sb * 56 + u * 7 + s
                k = jnp.minimum(q // HIST, NUM_TABLES - 1)
                slot = k - k0
                live = q < NLOOK
                w0 = jnp.where(live & (slot == 0), 1.0, 0.0)
                w1 = jnp.where(live & (slot == 1), 1.0, 0.0)
                w0v = jnp.full((LANES,), w0, jnp.float32)
                w1v = jnp.full((LANES,), w1, jnp.float32)
                lanev = jnp.full((LANES,), lane, jnp.int32)
                slotv = jnp.full((LANES,), s, jnp.int32)
                new = []
                for j in range(DIM // LANES):
                    g_ = plsc.load_gather(
                        blk, [pv, slotv, iota16 + 16 * j, lanev])
                    new.append(acc[j] + g_ * w0v)
                    new.append(acc[DIM // LANES + j] + g_ * w1v)
                acc = tuple(new[::2]) + tuple(new[1::2])
            return acc

        fire(0, 0)

        def sbody(sb, acc):
            for u in range(8):
                if u < 7:
                    fire(sb, u + 1)
                else:
                    @pl.when(sb + 1 < NSB)
                    def _():
                        fire(sb + 1, 0)
                p = u % 2
                for s in range(GRP):
                    pltpu.make_async_copy(
                        table_hbm.at[k0, :, pl.ds(0, WIN)], blk.at[p, s],
                        sem.at[p]).wait()
                acc = extract(sb, u, acc)
            return acc

        acc = lax.fori_loop(
            0, NSB, sbody,
            tuple(jnp.zeros((LANES,), jnp.float32)
                  for _ in range(2 * (DIM // LANES))))
        for u in range(2):
            for j in range(DIM // LANES):
                ly_v[u, pl.ds(j * LANES, LANES)] = acc[u * (DIM // LANES) + j]
        pltpu.sync_copy(ly_v, out_hbm.at[t])


def _bags(idx_flat, emb):
    mesh = plsc.VectorSubcoreMesh(core_axis_name="c", subcore_axis_name="s")
    f = functools.partial(
        pl.kernel,
        out_type=jax.ShapeDtypeStruct((NTILES, 2, DIM), jnp.float32),
        mesh=mesh,
        scratch_types=[
            pltpu.VMEM((PER_TILE + LANES,), jnp.int32),
            pltpu.VMEM((2, GRP, DIM, WIN), jnp.float32),
            pltpu.VMEM((2, DIM), jnp.float32),
            pltpu.SemaphoreType.DMA((2,)),
        ],
        compiler_params=pltpu.CompilerParams(needs_layout_passes=False),
    )(_bags_body)
    return f(idx_flat, emb)


def _mv(W, x):
    return lax.dot_general(W, x, (((1,), (0,)), ((), ())),
                           preferred_element_type=jnp.float32)


def _mlp_body(x_ref, parts_ref, comb_ref, sx_ref, a2_ref, b2t_ref,
              bw0, bb0, bw1, bb1, bw2, bb2,
              tw0x, tw0z, tb0, tw1, tb1, tw2, tb2, out_ref):
    x = x_ref[...]
    x = jnp.maximum(_mv(bw0[...], x) + bb0[...], 0.0)
    x = jnp.maximum(_mv(bw1[...], x) + bb1[...], 0.0)
    x = jnp.maximum(_mv(bw2[...], x) + bb2[...], 0.0)

    L = lax.dot_general(comb_ref[...], parts_ref[...], (((1,), (0,)), ((), ())),
                        preferred_element_type=jnp.float32)
    zx = _mv(L, x)
    ZL = lax.dot_general(L, L, (((1,), (1,)), ((), ())),
                         preferred_element_type=jnp.float32)
    zf = _mv(sx_ref[...], zx)
    ZA = lax.dot_general(a2_ref[...], ZL, (((1,), (0,)), ((), ())),
                         preferred_element_type=jnp.float32)
    zf = zf + jnp.sum(ZA * b2t_ref[...], axis=1, keepdims=True)

    h = jnp.maximum(_mv(tw0x[...], x) + _mv(tw0z[...], zf) + tb0[...], 0.0)
    h = jnp.maximum(_mv(tw1[...], h) + tb1[...], 0.0)
    h = _mv(tw2[...], h) + tb2[...]
    out_ref[...] = jax.nn.sigmoid(h)


def kernel(dense_x, lS_i, emb, bot_Ws, bot_bs, top_Ws, top_bs):
    idx_flat = jnp.pad(lS_i[:, 0, :].reshape(-1), (0, IDX_PAD - NLOOK))
    parts = _bags(idx_flat, emb.transpose(0, 2, 1))
    parts = parts.reshape(2 * NTILES, DIM)

    x0 = dense_x.reshape(13, 1)
    comb = jnp.asarray(_COMB)
    sx = jnp.asarray(_SX)
    a2 = jnp.asarray(_A2)
    b2t = jnp.asarray(_B2T)
    bb = [b.reshape(-1, 1) for b in bot_bs]
    tb = [b.reshape(-1, 1) for b in top_bs]
    tw0x = top_Ws[0][:, :DIM]
    tw0z = top_Ws[0][:, DIM:]

    return pl.pallas_call(
        _mlp_body,
        out_shape=jax.ShapeDtypeStruct((1, 1), jnp.float32),
    )(x0, parts, comb, sx, a2, b2t,
      bot_Ws[0], bb[0], bot_Ws[1], bb[1], bot_Ws[2], bb[2],
      tw0x, tw0z, tb[0], top_Ws[1], tb[1], top_Ws[2], tb[2])
